# baseline (speedup 1.0000x reference)
import jax
import numpy as np
import jax.numpy as jnp
from jax import lax
from jax.experimental import pallas as pl
from jax.experimental.pallas import tpu as pltpu

_P = 64
_B = 4096
_H = 128
_ALPHA = 0.5
_BBLK = 128
_TINY = np.float32(1.17549435e-38)
_SEED_LO = 1234


def _threefry_bits(x1):
    k0 = np.uint32(0)
    k1 = np.uint32(_SEED_LO)
    k2 = k0 ^ k1 ^ np.uint32(0x1BD11BDA)
    ks = (k0, k1, k2)
    rots = ((13, 15, 26, 6), (17, 29, 16, 24))

    x1 = x1.astype(jnp.uint32)

    def rotl(v, d):
        return (v << np.uint32(d)) | (v >> np.uint32(32 - d))

    x0 = x1
    x1 = rotl(x1, rots[0][0])
    x1 = x0 ^ x1
    for i in range(5):
        for r in rots[i % 2][1 if i == 0 else 0:]:
            x0 = x0 + x1
            x1 = rotl(x1, r)
            x1 = x0 ^ x1
        if (i + 1) % 3 != 0:
            x0 = x0 + ks[(i + 1) % 3]
        x1 = x1 + np.uint32(ks[(i + 2) % 3] + np.uint32(i + 1))
    return x0 ^ x1


def _prep_kernel(prob2d_ref, p64c_ref, invr_ref, wcol_ref, c2d_ref):
    resamp = _ALPHA * jnp.exp(prob2d_ref[...]) + (1.0 - _ALPHA) / _P
    invr_ref[...] = np.float32(1.0) / resamp
    pn = jnp.exp(p64c_ref[...])
    wcol_ref[...] = jnp.log(pn / (_ALPHA * pn + (1.0 - _ALPHA) / _P))
    ps_io = lax.broadcasted_iota(jnp.uint32, (_P, _B), 0)
    b_io = lax.broadcasted_iota(jnp.uint32, (_P, _B), 1)
    c2d_ref[...] = (b_io * np.uint32(_P * _P) + ps_io * np.uint32(_P)
                    + np.uint32(_SEED_LO))


def _lse_kernel(wg_ref, pnew_ref):
    wg = wg_ref[...]
    m2 = jnp.max(wg, axis=0, keepdims=True)
    lse = m2 + jnp.log(jnp.sum(jnp.exp(wg - m2), axis=0, keepdims=True))
    pnew_ref[...] = wg - lse


_NSPLIT = 4


def _fused_kernel(invr_ref, c2d_ref, wcol_ref, table_ref, out_ref, wg_ref):
    ph = _P // _NSPLIT
    invr = invr_ref[...]
    for s in range(_NSPLIT):
        shp = (ph, _P, _BBLK)

        pp_io = lax.broadcasted_iota(jnp.uint32, shp, 1)
        x1 = c2d_ref[s * ph:(s + 1) * ph, :][:, None, :] + pp_io

        bits = _threefry_bits(x1)
        fb = (bits >> np.uint32(9)) | np.uint32(0x3F800000)
        u = lax.bitcast_convert_type(fb, jnp.float32) - np.float32(1.0)
        logu = jnp.log(u)

        v = logu * invr[None, :, :]

        m = jnp.max(v, axis=1, keepdims=True)
        pidx = lax.broadcasted_iota(jnp.int32, shp, 1)
        sidx = jnp.min(jnp.where(v == m, pidx, _P), axis=1, keepdims=True)
        idx_t = jnp.swapaxes(sidx, 1, 2)

        lane = lax.broadcasted_iota(jnp.int32, (ph, _BBLK, _P), 2)
        oh = (lane == idx_t).astype(jnp.float32)

        out_ref[s * ph:(s + 1) * ph, :, :] = jnp.dot(
            oh.reshape(ph * _BBLK, _P), table_ref[...],
            preferred_element_type=jnp.float32,
        ).reshape(ph, _BBLK, _H)

        wg8 = jnp.dot(
            oh.reshape(ph * _BBLK, _P), wcol_ref[...],
            preferred_element_type=jnp.float32,
        ).reshape(ph, _BBLK, 8)
        wg_ref[s * ph:(s + 1) * ph, :] = jnp.max(wg8, axis=2)


@jax.jit
def kernel(particles, prob):
    prob2d = prob.reshape(_P, _B)
    p64c = jnp.broadcast_to(prob.reshape(-1)[:_P].reshape(_P, 1), (_P, 8))
    table = particles[:_P, :]

    invr, wcol, c2d = pl.pallas_call(
        _prep_kernel,
        out_shape=[
            jax.ShapeDtypeStruct((_P, _B), jnp.float32),
            jax.ShapeDtypeStruct((_P, 8), jnp.float32),
            jax.ShapeDtypeStruct((_P, _B), jnp.uint32),
        ],
    )(prob2d, p64c)

    out, wg = pl.pallas_call(
        _fused_kernel,
        grid=(_B // _BBLK,),
        compiler_params=pltpu.CompilerParams(
            dimension_semantics=("parallel",),
            vmem_limit_bytes=128 * 1024 * 1024),
        in_specs=[
            pl.BlockSpec((_P, _BBLK), lambda ib: (0, ib)),
            pl.BlockSpec((_P, _BBLK), lambda ib: (0, ib)),
            pl.BlockSpec((_P, 8), lambda ib: (0, 0)),
            pl.BlockSpec((_P, _H), lambda ib: (0, 0)),
        ],
        out_specs=[
            pl.BlockSpec((_P, _BBLK, _H), lambda ib: (0, ib, 0)),
            pl.BlockSpec((_P, _BBLK), lambda ib: (0, ib)),
        ],
        out_shape=[
            jax.ShapeDtypeStruct((_P, _B, _H), jnp.float32),
            jax.ShapeDtypeStruct((_P, _B), jnp.float32),
        ],
    )(invr, c2d, wcol, table)

    pnew = pl.pallas_call(
        _lse_kernel,
        out_shape=jax.ShapeDtypeStruct((_P, _B), jnp.float32),
    )(wg)
    return out.reshape(_P * _B, _H), pnew


# speedup vs baseline: 1.0551x; 1.0551x over previous
import jax
import numpy as np
import jax.numpy as jnp
from jax import lax
from jax.experimental import pallas as pl
from jax.experimental.pallas import tpu as pltpu

_P = 64
_B = 4096
_H = 128
_ALPHA = 0.5
_BBLK = 128
_SEED_LO = 1234


def _threefry_bits(x1):
    k0 = np.uint32(0)
    k1 = np.uint32(_SEED_LO)
    k2 = k0 ^ k1 ^ np.uint32(0x1BD11BDA)
    ks = (k0, k1, k2)
    rots = ((13, 15, 26, 6), (17, 29, 16, 24))

    x1 = x1.astype(jnp.uint32)

    def rotl(v, d):
        return (v << np.uint32(d)) | (v >> np.uint32(32 - d))

    x0 = x1
    x1 = rotl(x1, rots[0][0])
    x1 = x0 ^ x1
    for i in range(5):
        for r in rots[i % 2][1 if i == 0 else 0:]:
            x0 = x0 + x1
            x1 = rotl(x1, r)
            x1 = x0 ^ x1
        if (i + 1) % 3 != 0:
            x0 = x0 + ks[(i + 1) % 3]
        x1 = x1 + np.uint32(ks[(i + 2) % 3] + np.uint32(i + 1))
    return x0 ^ x1


def _prep_kernel(prob2d_ref, p64c_ref, invr_ref, wcol_ref, c2d_ref):
    resamp = _ALPHA * jnp.exp(prob2d_ref[...]) + (1.0 - _ALPHA) / _P
    invr_ref[...] = np.float32(1.0) / resamp
    pn = jnp.exp(p64c_ref[...])
    wcol_ref[...] = jnp.log(pn / (_ALPHA * pn + (1.0 - _ALPHA) / _P))
    ps_io = lax.broadcasted_iota(jnp.uint32, (_P, _B), 0)
    b_io = lax.broadcasted_iota(jnp.uint32, (_P, _B), 1)
    c2d_ref[...] = (b_io * np.uint32(_P * _P) + ps_io * np.uint32(_P)
                    + np.uint32(_SEED_LO))


def _lse_kernel(wg_ref, pnew_ref):
    wg = wg_ref[...]
    m2 = jnp.max(wg, axis=0, keepdims=True)
    lse = m2 + jnp.log(jnp.sum(jnp.exp(wg - m2), axis=0, keepdims=True))
    pnew_ref[...] = wg - lse


_NSPLIT = 2


def _fused_kernel(invr_ref, c2d_ref, wcol_ref, table_ref, out_ref, wg_ref):
    ph = _P // _NSPLIT
    invr = invr_ref[...]
    for s in range(_NSPLIT):
        shp = (ph, _P, _BBLK)

        pp_io = lax.broadcasted_iota(jnp.uint32, shp, 1)
        x1 = c2d_ref[s * ph:(s + 1) * ph, :][:, None, :] + pp_io

        bits = _threefry_bits(x1)
        fb = (bits >> np.uint32(9)) | np.uint32(0x3F800000)
        u = lax.bitcast_convert_type(fb, jnp.float32) - np.float32(1.0)
        logu = jnp.log(u)

        v = logu * invr[None, :, :]

        m = jnp.max(v, axis=1, keepdims=True)
        pidx = lax.broadcasted_iota(jnp.int32, shp, 1)
        sidx = jnp.min(jnp.where(v == m, pidx, _P), axis=1, keepdims=True)
        idx_t = jnp.swapaxes(sidx, 1, 2)

        lane = lax.broadcasted_iota(jnp.int32, (ph, _BBLK, _P), 2)
        oh = (lane == idx_t).astype(jnp.float32)

        out_ref[s * ph:(s + 1) * ph, :, :] = jnp.dot(
            oh.reshape(ph * _BBLK, _P), table_ref[...],
            preferred_element_type=jnp.float32,
        ).reshape(ph, _BBLK, _H)

        wg8 = jnp.dot(
            oh.reshape(ph * _BBLK, _P), wcol_ref[...],
            preferred_element_type=jnp.float32,
        ).reshape(ph, _BBLK, 8)
        wg_ref[s * ph:(s + 1) * ph, :] = jnp.max(wg8, axis=2)


@jax.jit
def kernel(particles, prob):
    prob2d = prob.reshape(_P, _B)
    p64c = jnp.broadcast_to(prob.reshape(-1)[:_P].reshape(_P, 1), (_P, 8))
    table = particles[:_P, :]

    invr, wcol, c2d = pl.pallas_call(
        _prep_kernel,
        out_shape=[
            jax.ShapeDtypeStruct((_P, _B), jnp.float32),
            jax.ShapeDtypeStruct((_P, 8), jnp.float32),
            jax.ShapeDtypeStruct((_P, _B), jnp.uint32),
        ],
    )(prob2d, p64c)

    out, wg = pl.pallas_call(
        _fused_kernel,
        grid=(_B // _BBLK,),
        compiler_params=pltpu.CompilerParams(
            dimension_semantics=("parallel",),
            vmem_limit_bytes=128 * 1024 * 1024),
        in_specs=[
            pl.BlockSpec((_P, _BBLK), lambda ib: (0, ib)),
            pl.BlockSpec((_P, _BBLK), lambda ib: (0, ib)),
            pl.BlockSpec((_P, 8), lambda ib: (0, 0)),
            pl.BlockSpec((_P, _H), lambda ib: (0, 0)),
        ],
        out_specs=[
            pl.BlockSpec((_P, _BBLK, _H), lambda ib: (0, ib, 0)),
            pl.BlockSpec((_P, _BBLK), lambda ib: (0, ib)),
        ],
        out_shape=[
            jax.ShapeDtypeStruct((_P, _B, _H), jnp.float32),
            jax.ShapeDtypeStruct((_P, _B), jnp.float32),
        ],
    )(invr, c2d, wcol, table)

    pnew = pl.pallas_call(
        _lse_kernel,
        out_shape=jax.ShapeDtypeStruct((_P, _B), jnp.float32),
    )(wg)
    return out.reshape(_P * _B, _H), pnew
